# baseline (speedup 1.0000x reference)
import functools

import jax
import jax.numpy as jnp
from jax import lax
from jax.experimental import pallas as pl
from jax.experimental.pallas import tpu as pltpu

_TM = 256
_B = 2048


def _round_up(v, m):
    return (v + m - 1) // m * m


def _pad2(w, rows, cols):
    return jnp.zeros((rows, cols), jnp.float32).at[: w.shape[0], : w.shape[1]].set(w)


def _deg_body(blk_lo_ref, blk_hi_ref, dst_ref, wv_ref, o_ref):
    t = pl.program_id(0)
    base = t * _TM
    lo = blk_lo_ref[t]
    hi = blk_hi_ref[t]
    row_ids = base + lax.broadcasted_iota(jnp.int32, (_TM, _B), 0)

    def blk(j, acc):
        k = lo + j
        onehot = jnp.where(dst_ref[k] == row_ids,
                           jnp.broadcast_to(wv_ref[k], (_TM, _B)), 0.0)
        return acc + jnp.sum(onehot, axis=1, keepdims=True)

    deg = lax.fori_loop(0, hi - lo, blk, jnp.zeros((_TM, 1), jnp.float32))
    o_ref[...] = jnp.broadcast_to(deg, (_TM, 128))


def _body(h_pad, c_pad, blk_lo_ref, blk_hi_ref, srcw_ref,
          dst_ref, wv_ref, x_ref, xt_ref, dinv_ref, wc_ref, bc_ref,
          lz_ref, lh_ref, lbz_ref, lbh_ref, wr_ref, br_ref, o_ref,
          gt_ref, ax_ref):
    t = pl.program_id(0)
    base = t * _TM
    lo = blk_lo_ref[t]
    hi = blk_hi_ref[t]
    s = _B + 1
    p = c_pad // 128

    ax_ref[...] = xt_ref[...]

    row_ids = base + lax.broadcasted_iota(jnp.int32, (_TM, _B), 0)

    def blk(j, carry):
        k = lo + j
        kb = k * _B
        for mi in range(_B):
            idx = pl.multiple_of(srcw_ref[kb + mi], p)
            gt_ref[mi:mi + p * s:s, :] = x_ref[pl.ds(idx, p), :]
        g = jnp.concatenate(
            [gt_ref[pl.ds(c * s, _B), :] for c in range(p)], axis=-1)
        dstv = dst_ref[k]
        wv = wv_ref[k]
        onehot = jnp.where(dstv == row_ids,
                           jnp.broadcast_to(wv, (_TM, _B)), 0.0)
        ax_ref[...] += jnp.dot(onehot, g, preferred_element_type=jnp.float32)
        return carry

    lax.fori_loop(0, hi - lo, blk, 0)

    ax = ax_ref[...] * dinv_ref[:, :1]
    conv = jnp.dot(ax, wc_ref[...], preferred_element_type=jnp.float32) + bc_ref[...]
    g_z = jnp.dot(conv[:, :h_pad], lz_ref[...],
                  preferred_element_type=jnp.float32) + lbz_ref[...]
    g_h = jnp.dot(conv[:, h_pad:], lh_ref[...],
                  preferred_element_type=jnp.float32) + lbh_ref[...]
    hn = (1.0 - jax.nn.sigmoid(g_z)) * jnp.tanh(g_h)
    o_ref[...] = jnp.dot(hn, wr_ref[...],
                         preferred_element_type=jnp.float32) + br_ref[...]


def kernel(x, edge_index, edge_attr, w_z, w_r, w_h, b_z, b_r, b_h,
           lz1, lz2, lr1, lr2, lh1, lh2, lb_z, lb_r, lb_h, w_reg, b_reg):
    n, c = x.shape
    hidden = w_z.shape[1]
    out_ch = w_reg.shape[1]

    n_pad = _round_up(n, _TM)
    c_pad = _round_up(c, 128)
    h_pad = _round_up(hidden, 128)
    o_pad = _round_up(out_ch, 128)
    n_tiles = n_pad // _TM
    p = c_pad // 128

    src_a = edge_index[0]
    dst_a = edge_index[1]
    w_all = edge_attr

    e_ids = jnp.arange(src_a.shape[0], dtype=jnp.int32)
    srcw = ((jax.lax.bitcast_convert_type(w_all, jnp.int32) >> 13) << 13) | src_a
    packed, srcw_s = jax.lax.sort(((dst_a << 17) | e_ids, srcw), num_keys=1)
    dst_s = packed >> 17

    e = dst_s.shape[0]
    e_pad = _round_up(e, _B)
    n_blk = e_pad // _B
    pad = e_pad - e
    dst_s = jnp.concatenate([dst_s, jnp.full((pad,), -1, jnp.int32)])
    srcw_s = jnp.concatenate([srcw_s, jnp.zeros((pad,), jnp.int32)])
    src_s = (srcw_s & 8191) * p
    w_v = jax.lax.bitcast_convert_type((srcw_s >> 13) << 13, jnp.float32)

    tile_bounds = jnp.searchsorted(
        dst_s[:e], jnp.arange(n_tiles + 1, dtype=jnp.int32) * _TM).astype(jnp.int32)
    blk_lo = tile_bounds[:-1] // _B
    blk_hi = -((-tile_bounds[1:]) // _B)

    dst_v = dst_s.reshape(n_blk, 1, _B)
    wv_v = w_v.reshape(n_blk, 1, _B)

    deg_b = pl.pallas_call(
        _deg_body,
        out_shape=jax.ShapeDtypeStruct((n_pad, 128), jnp.float32),
        grid_spec=pltpu.PrefetchScalarGridSpec(
            num_scalar_prefetch=2,
            grid=(n_tiles,),
            in_specs=[
                pl.BlockSpec((n_blk, 1, _B), lambda i, *_: (0, 0, 0)),
                pl.BlockSpec((n_blk, 1, _B), lambda i, *_: (0, 0, 0)),
            ],
            out_specs=pl.BlockSpec((_TM, 128), lambda i, *_: (i, 0)),
        ),
        compiler_params=pltpu.CompilerParams(
            dimension_semantics=("parallel",)),
    )(blk_lo, blk_hi, dst_v, wv_v)

    deg_b = deg_b + 1.0
    dinv_b = jnp.where(deg_b > 0.0, deg_b ** -0.5, 0.0)
    xs2 = jnp.zeros((n_pad, c_pad), jnp.float32).at[:n, :c].set(
        x * dinv_b[:n, :1])
    xs_r = xs2.reshape(n_pad * p, 128)

    w_conv = jnp.concatenate(
        [_pad2(w_z, c_pad, h_pad), _pad2(w_h, c_pad, h_pad)], axis=1)
    b_conv = jnp.concatenate(
        [_pad2(b_z, 1, h_pad), _pad2(b_h, 1, h_pad)], axis=1)
    lz_p = _pad2(lz1, h_pad, h_pad)
    lh_p = _pad2(lh1, h_pad, h_pad)
    lbz_p = _pad2(lb_z, 1, h_pad)
    lbh_p = _pad2(lb_h, 1, h_pad)
    wr_p = _pad2(w_reg, h_pad, o_pad)
    br_p = _pad2(b_reg, 1, o_pad)

    def full(shape):
        return pl.BlockSpec(shape, lambda i, *_: (0,) * len(shape))

    flops = 2 * e_pad * _TM * c_pad + 2 * n_pad * (
        c_pad * 2 * h_pad + 2 * h_pad * h_pad + h_pad * o_pad)
    cost = pl.CostEstimate(
        flops=flops, transcendentals=2 * n_pad * h_pad,
        bytes_accessed=4 * (e_pad * (2 + c_pad) + n_pad * c_pad
                            + n_pad * o_pad))

    out_pad = pl.pallas_call(
        functools.partial(_body, h_pad, c_pad),
        out_shape=jax.ShapeDtypeStruct((n_pad, o_pad), jnp.float32),
        grid_spec=pltpu.PrefetchScalarGridSpec(
            num_scalar_prefetch=3,
            grid=(n_tiles,),
            in_specs=[
                full((n_blk, 1, _B)),
                full((n_blk, 1, _B)),
                full((n_pad * p, 128)),
                pl.BlockSpec((_TM, c_pad), lambda i, *_: (i, 0)),
                pl.BlockSpec((_TM, 128), lambda i, *_: (i, 0)),
                full((c_pad, 2 * h_pad)),
                full((1, 2 * h_pad)),
                full((h_pad, h_pad)),
                full((h_pad, h_pad)),
                full((1, h_pad)),
                full((1, h_pad)),
                full((h_pad, o_pad)),
                full((1, o_pad)),
            ],
            out_specs=pl.BlockSpec((_TM, o_pad), lambda i, *_: (i, 0)),
            scratch_shapes=[
                pltpu.VMEM((p * (_B + 1), 128), jnp.float32),
                pltpu.VMEM((_TM, c_pad), jnp.float32),
            ],
        ),
        compiler_params=pltpu.CompilerParams(
            dimension_semantics=("parallel",)),
        cost_estimate=cost,
    )(blk_lo, blk_hi, src_s, dst_v, wv_v, xs_r, xs2, dinv_b,
      w_conv, b_conv, lz_p, lh_p, lbz_p, lbh_p, wr_p, br_p)

    return out_pad[:n, :out_ch]


# speedup vs baseline: 1.0842x; 1.0842x over previous
import functools

import jax
import jax.numpy as jnp
from jax import lax
from jax.experimental import pallas as pl
from jax.experimental.pallas import tpu as pltpu

_TM = 256
_B = 1024


def _round_up(v, m):
    return (v + m - 1) // m * m


def _pad2(w, rows, cols):
    return jnp.zeros((rows, cols), jnp.float32).at[: w.shape[0], : w.shape[1]].set(w)


def _deg_body(blk_lo_ref, blk_hi_ref, dst_ref, wv_ref, o_ref):
    t = pl.program_id(0)
    base = t * _TM
    lo = blk_lo_ref[t]
    hi = blk_hi_ref[t]
    row_ids = base + lax.broadcasted_iota(jnp.int32, (_TM, _B), 0)

    def blk(j, acc):
        k = lo + j
        onehot = jnp.where(dst_ref[k] == row_ids,
                           jnp.broadcast_to(wv_ref[k], (_TM, _B)), 0.0)
        return acc + jnp.sum(onehot, axis=1, keepdims=True)

    deg = lax.fori_loop(0, hi - lo, blk, jnp.zeros((_TM, 1), jnp.float32))
    o_ref[...] = jnp.broadcast_to(deg, (_TM, 128))


def _body(h_pad, c_pad, blk_lo_ref, blk_hi_ref, srcw_ref,
          dst_ref, wv_ref, x_ref, xt_ref, dinv_ref, wc_ref, bc_ref,
          lz_ref, lh_ref, lbz_ref, lbh_ref, wr_ref, br_ref, o_ref,
          gt_ref, ax_ref):
    t = pl.program_id(0)
    base = t * _TM
    lo = blk_lo_ref[t]
    hi = blk_hi_ref[t]
    s = _B + 1
    p = c_pad // 128

    ax_ref[...] = xt_ref[...]

    row_ids = base + lax.broadcasted_iota(jnp.int32, (_TM, _B), 0)

    def blk(j, carry):
        k = lo + j
        kb = k * _B
        for mi in range(_B):
            idx = pl.multiple_of(srcw_ref[kb + mi], p)
            gt_ref[mi:mi + p * s:s, :] = x_ref[pl.ds(idx, p), :]
        g = jnp.concatenate(
            [gt_ref[pl.ds(c * s, _B), :] for c in range(p)], axis=-1)
        dstv = dst_ref[k]
        wv = wv_ref[k]
        onehot = jnp.where(dstv == row_ids,
                           jnp.broadcast_to(wv, (_TM, _B)), 0.0)
        ax_ref[...] += jnp.dot(onehot, g, preferred_element_type=jnp.float32)
        return carry

    lax.fori_loop(0, hi - lo, blk, 0)

    ax = ax_ref[...] * dinv_ref[:, :1]
    conv = jnp.dot(ax, wc_ref[...], preferred_element_type=jnp.float32) + bc_ref[...]
    g_z = jnp.dot(conv[:, :h_pad], lz_ref[...],
                  preferred_element_type=jnp.float32) + lbz_ref[...]
    g_h = jnp.dot(conv[:, h_pad:], lh_ref[...],
                  preferred_element_type=jnp.float32) + lbh_ref[...]
    hn = (1.0 - jax.nn.sigmoid(g_z)) * jnp.tanh(g_h)
    o_ref[...] = jnp.dot(hn, wr_ref[...],
                         preferred_element_type=jnp.float32) + br_ref[...]


def kernel(x, edge_index, edge_attr, w_z, w_r, w_h, b_z, b_r, b_h,
           lz1, lz2, lr1, lr2, lh1, lh2, lb_z, lb_r, lb_h, w_reg, b_reg):
    n, c = x.shape
    hidden = w_z.shape[1]
    out_ch = w_reg.shape[1]

    n_pad = _round_up(n, _TM)
    c_pad = _round_up(c, 128)
    h_pad = _round_up(hidden, 128)
    o_pad = _round_up(out_ch, 128)
    n_tiles = n_pad // _TM
    p = c_pad // 128

    src_a = edge_index[0]
    dst_a = edge_index[1]
    w_all = edge_attr

    e_ids = jnp.arange(src_a.shape[0], dtype=jnp.int32)
    srcw = ((jax.lax.bitcast_convert_type(w_all, jnp.int32) >> 13) << 13) | src_a
    packed, srcw_s = jax.lax.sort(((dst_a << 17) | e_ids, srcw), num_keys=1)
    dst_s = packed >> 17

    e = dst_s.shape[0]
    e_pad = _round_up(e, _B)
    n_blk = e_pad // _B
    pad = e_pad - e
    dst_s = jnp.concatenate([dst_s, jnp.full((pad,), -1, jnp.int32)])
    srcw_s = jnp.concatenate([srcw_s, jnp.zeros((pad,), jnp.int32)])
    src_s = (srcw_s & 8191) * p
    w_v = jax.lax.bitcast_convert_type((srcw_s >> 13) << 13, jnp.float32)

    tile_bounds = jnp.searchsorted(
        dst_s[:e], jnp.arange(n_tiles + 1, dtype=jnp.int32) * _TM).astype(jnp.int32)
    blk_lo = tile_bounds[:-1] // _B
    blk_hi = -((-tile_bounds[1:]) // _B)

    dst_v = dst_s.reshape(n_blk, 1, _B)
    wv_v = w_v.reshape(n_blk, 1, _B)

    deg_b = pl.pallas_call(
        _deg_body,
        out_shape=jax.ShapeDtypeStruct((n_pad, 128), jnp.float32),
        grid_spec=pltpu.PrefetchScalarGridSpec(
            num_scalar_prefetch=2,
            grid=(n_tiles,),
            in_specs=[
                pl.BlockSpec((n_blk, 1, _B), lambda i, *_: (0, 0, 0)),
                pl.BlockSpec((n_blk, 1, _B), lambda i, *_: (0, 0, 0)),
            ],
            out_specs=pl.BlockSpec((_TM, 128), lambda i, *_: (i, 0)),
        ),
        compiler_params=pltpu.CompilerParams(
            dimension_semantics=("parallel",)),
    )(blk_lo, blk_hi, dst_v, wv_v)

    deg_b = deg_b + 1.0
    dinv_b = jnp.where(deg_b > 0.0, deg_b ** -0.5, 0.0)
    xs2 = jnp.zeros((n_pad, c_pad), jnp.float32).at[:n, :c].set(
        x * dinv_b[:n, :1])
    xs_r = xs2.reshape(n_pad * p, 128)

    w_conv = jnp.concatenate(
        [_pad2(w_z, c_pad, h_pad), _pad2(w_h, c_pad, h_pad)], axis=1)
    b_conv = jnp.concatenate(
        [_pad2(b_z, 1, h_pad), _pad2(b_h, 1, h_pad)], axis=1)
    lz_p = _pad2(lz1, h_pad, h_pad)
    lh_p = _pad2(lh1, h_pad, h_pad)
    lbz_p = _pad2(lb_z, 1, h_pad)
    lbh_p = _pad2(lb_h, 1, h_pad)
    wr_p = _pad2(w_reg, h_pad, o_pad)
    br_p = _pad2(b_reg, 1, o_pad)

    def full(shape):
        return pl.BlockSpec(shape, lambda i, *_: (0,) * len(shape))

    flops = 2 * e_pad * _TM * c_pad + 2 * n_pad * (
        c_pad * 2 * h_pad + 2 * h_pad * h_pad + h_pad * o_pad)
    cost = pl.CostEstimate(
        flops=flops, transcendentals=2 * n_pad * h_pad,
        bytes_accessed=4 * (e_pad * (2 + c_pad) + n_pad * c_pad
                            + n_pad * o_pad))

    out_pad = pl.pallas_call(
        functools.partial(_body, h_pad, c_pad),
        out_shape=jax.ShapeDtypeStruct((n_pad, o_pad), jnp.float32),
        grid_spec=pltpu.PrefetchScalarGridSpec(
            num_scalar_prefetch=3,
            grid=(n_tiles,),
            in_specs=[
                full((n_blk, 1, _B)),
                full((n_blk, 1, _B)),
                full((n_pad * p, 128)),
                pl.BlockSpec((_TM, c_pad), lambda i, *_: (i, 0)),
                pl.BlockSpec((_TM, 128), lambda i, *_: (i, 0)),
                full((c_pad, 2 * h_pad)),
                full((1, 2 * h_pad)),
                full((h_pad, h_pad)),
                full((h_pad, h_pad)),
                full((1, h_pad)),
                full((1, h_pad)),
                full((h_pad, o_pad)),
                full((1, o_pad)),
            ],
            out_specs=pl.BlockSpec((_TM, o_pad), lambda i, *_: (i, 0)),
            scratch_shapes=[
                pltpu.VMEM((p * (_B + 1), 128), jnp.float32),
                pltpu.VMEM((_TM, c_pad), jnp.float32),
            ],
        ),
        compiler_params=pltpu.CompilerParams(
            dimension_semantics=("parallel",)),
        cost_estimate=cost,
    )(blk_lo, blk_hi, src_s, dst_v, wv_v, xs_r, xs2, dinv_b,
      w_conv, b_conv, lz_p, lh_p, lbz_p, lbh_p, wr_p, br_p)

    return out_pad[:n, :out_ch]
